# baseline (speedup 1.0000x reference)
import functools

import jax
import jax.numpy as jnp
from jax import lax
from jax.experimental import pallas as pl
from jax.experimental.pallas import tpu as pltpu
from jax.experimental.pallas import tpu_sc as plsc

N_FIELDS = 26
VOCAB = 100000
EMBED_DIM = 32
BATCH = 16384

NUM_CORES = 2
NUM_SUBCORES = 16
IDX_CHUNK = 4096
N_IDX_CHUNKS = BATCH // IDX_CHUNK
N_UNITS = N_FIELDS * N_IDX_CHUNKS

_mesh = plsc.VectorSubcoreMesh(
    core_axis_name="c", subcore_axis_name="s",
    num_cores=NUM_CORES, num_subcores=NUM_SUBCORES)


@functools.partial(
    pl.kernel,
    mesh=_mesh,
    out_type=jax.ShapeDtypeStruct((EMBED_DIM, BATCH), jnp.float32),
    scratch_types=[
        pltpu.VMEM((VOCAB,), jnp.float32),
        pltpu.VMEM((2, IDX_CHUNK), jnp.int32),
        pltpu.VMEM((BATCH,), jnp.float32),
        pltpu.VMEM_SHARED((2, BATCH), jnp.int32),
        pltpu.SemaphoreType.DMA,
        pltpu.SemaphoreType.DMA,
        pltpu.SemaphoreType.DMA,
    ],
    compiler_params=pltpu.CompilerParams(use_tc_tiling_on_sc=True,
                                         needs_layout_passes=False),
)
def _emb_sum_t(tt_hbm, xt_hbm, out_hbm, stripe_v, idx_v, acc_v, idx_sh,
               sem_s, sem_i, sem_b):
    sid = lax.axis_index("s")
    w = sid * NUM_CORES + lax.axis_index("c")
    d = w

    @pl.when(sid == 0)
    def _():
        pltpu.async_copy(xt_hbm.at[0], idx_sh.at[0], sem_b)

    def issue_pull(u):
        f, h = u // N_IDX_CHUNKS, u % N_IDX_CHUNKS
        return pltpu.async_copy(
            idx_sh.at[f % 2, pl.ds(h * IDX_CHUNK, IDX_CHUNK)],
            idx_v.at[u % 2], sem_i)

    def unit_compute(f, h, buf):
        def vreg_body(i):
            iv = idx_v[buf, pl.ds(i * 16, 16)]
            g = plsc.load_gather(stripe_v, [iv])
            o = h * IDX_CHUNK + i * 16
            acc_v[pl.ds(o, 16)] = acc_v[pl.ds(o, 16)] + g

        plsc.parallel_loop(0, IDX_CHUNK // 16, unroll=8)(vreg_body)

    def zero_body(i):
        acc_v[pl.ds(i * 16, 16)] = jnp.zeros((16,), jnp.float32)

    plsc.parallel_loop(0, BATCH // 16, unroll=8)(zero_body)

    def field_body(f, carry):
        @pl.when(sid == 0)
        def _():
            pltpu.make_async_copy(xt_hbm.at[f], idx_sh.at[f % 2],
                                  sem_b).wait()

        plsc.subcore_barrier()

        @pl.when((sid == 0) & (f + 1 < N_FIELDS))
        def _():
            pltpu.async_copy(xt_hbm.at[f + 1], idx_sh.at[(f + 1) % 2], sem_b)

        stripe_copy = pltpu.async_copy(tt_hbm.at[f, d], stripe_v, sem_s)
        issue_pull(f * N_IDX_CHUNKS)
        stripe_copy.wait()

        def chunk_body(h, carry2):
            u = f * N_IDX_CHUNKS + h

            @pl.when(h + 1 < N_IDX_CHUNKS)
            def _():
                issue_pull(u + 1)

            pltpu.make_async_copy(
                idx_sh.at[f % 2, pl.ds(h * IDX_CHUNK, IDX_CHUNK)],
                idx_v.at[u % 2], sem_i).wait()

            unit_compute(f, h, u % 2)
            return carry2

        lax.fori_loop(0, N_IDX_CHUNKS, chunk_body, 0, unroll=True)
        return carry

    lax.fori_loop(0, N_FIELDS, field_body, 0)

    pltpu.sync_copy(acc_v, out_hbm.at[d])


def kernel(x, tables):
    tt = tables.transpose(0, 2, 1)
    xt = x.T
    out_t = _emb_sum_t(tt, xt)
    return out_t.T


# speedup vs baseline: 1.2410x; 1.2410x over previous
import functools

import jax
import jax.numpy as jnp
from jax import lax
from jax.experimental import pallas as pl
from jax.experimental.pallas import tpu as pltpu
from jax.experimental.pallas import tpu_sc as plsc

N_FIELDS = 26
VOCAB = 100000
EMBED_DIM = 32
BATCH = 16384

NUM_CORES = 2
NUM_SUBCORES = 16
IDX_CHUNK = 4096
N_IDX_CHUNKS = BATCH // IDX_CHUNK

SPLIT = 50048
TAIL_START = 99872
TAIL = 128
HALF_LENS = (SPLIT, VOCAB - SPLIT)
HALF_DMA_LENS = (SPLIT, 49920)

_mesh = plsc.VectorSubcoreMesh(
    core_axis_name="c", subcore_axis_name="s",
    num_cores=NUM_CORES, num_subcores=NUM_SUBCORES)


@functools.partial(
    pl.kernel,
    mesh=_mesh,
    out_type=jax.ShapeDtypeStruct((EMBED_DIM, BATCH), jnp.float32),
    scratch_types=[
        pltpu.VMEM((SPLIT,), jnp.float32),
        pltpu.VMEM((SPLIT,), jnp.float32),
        pltpu.VMEM((2, IDX_CHUNK), jnp.int32),
        pltpu.VMEM((BATCH,), jnp.float32),
        pltpu.VMEM((N_FIELDS, TAIL), jnp.float32),
        pltpu.VMEM_SHARED((2, BATCH), jnp.int32),
        pltpu.SemaphoreType.DMA,
        pltpu.SemaphoreType.DMA,
        pltpu.SemaphoreType.DMA,
        pltpu.SemaphoreType.DMA,
    ],
    compiler_params=pltpu.CompilerParams(use_tc_tiling_on_sc=True,
                                         needs_layout_passes=False),
)
def _emb_sum_t(tt_hbm, xt_hbm, tail_hbm, out_hbm, stripe0_v, stripe1_v,
               idx_v, acc_v, tail_v, idx_sh, sem_s0, sem_s1, sem_i, sem_b):
    sid = lax.axis_index("s")
    w = sid * NUM_CORES + lax.axis_index("c")
    d = w
    sems = (sem_s0, sem_s1)
    stripes = (stripe0_v, stripe1_v)

    def stripe_copies(f, h):
        off = h * SPLIT
        n = HALF_DMA_LENS[h]
        return [pltpu.make_async_copy(
            tt_hbm.at[f, :, pl.ds(off, n)].at[d],
            stripes[h].at[pl.ds(0, n)], sems[h])]

    def issue_stripe(f, h):
        for c in stripe_copies(f, h):
            c.start()

    def wait_stripe(f, h):
        for c in stripe_copies(f, h):
            c.wait()

    @pl.when(sid == 0)
    def _():
        pltpu.async_copy(xt_hbm.at[0], idx_sh.at[0], sem_b)

    def pull_copy(f, h, k):
        return pltpu.make_async_copy(
            idx_sh.at[f % 2, pl.ds(k * IDX_CHUNK, IDX_CHUNK)],
            idx_v.at[k % 2], sem_i)

    def half_pass(f, h):
        base = h * SPLIT
        bound = HALF_LENS[h]

        for k in range(N_IDX_CHUNKS):
            if k + 1 < N_IDX_CHUNKS:
                pull_copy(f, h, k + 1).start()
            pull_copy(f, h, k).wait()

            def vreg_body(i, k=k):
                iv = idx_v[k % 2, pl.ds(i * 16, 16)] - base
                m = plsc.bitcast(iv, jnp.uint32) < jnp.uint32(bound)
                g = plsc.load_gather(stripes[h], [iv], mask=m)
                g = jnp.where(m, g, jnp.float32(0.0))
                o = k * IDX_CHUNK + i * 16
                acc_v[pl.ds(o, 16)] = acc_v[pl.ds(o, 16)] + g

            plsc.parallel_loop(0, IDX_CHUNK // 16, unroll=8)(vreg_body)

    def zero_body(i):
        acc_v[pl.ds(i * 16, 16)] = jnp.zeros((16,), jnp.float32)

    plsc.parallel_loop(0, BATCH // 16, unroll=8)(zero_body)

    pltpu.sync_copy(tail_hbm.at[:, pl.ds(d * TAIL, TAIL)], tail_v)

    issue_stripe(0, 0)

    def field_body(f, carry):
        @pl.when(sid == 0)
        def _():
            pltpu.make_async_copy(xt_hbm.at[f], idx_sh.at[f % 2],
                                  sem_b).wait()

        plsc.subcore_barrier()

        @pl.when((sid == 0) & (f + 1 < N_FIELDS))
        def _():
            pltpu.async_copy(xt_hbm.at[f + 1], idx_sh.at[(f + 1) % 2], sem_b)

        wait_stripe(f, 0)
        issue_stripe(f, 1)
        pull_copy(f, 0, 0).start()
        half_pass(f, 0)

        wait_stripe(f, 1)
        for k in range(TAIL // 16):
            stripes[1][pl.ds(TAIL_START - SPLIT + k * 16, 16)] = (
                tail_v[f, pl.ds(k * 16, 16)])

        @pl.when(f + 1 < N_FIELDS)
        def _():
            issue_stripe(f + 1, 0)

        pull_copy(f, 1, 0).start()
        half_pass(f, 1)
        return carry

    lax.fori_loop(0, N_FIELDS, field_body, 0)

    pltpu.sync_copy(acc_v, out_hbm.at[d])


def kernel(x, tables):
    tt = tables.transpose(0, 2, 1)
    xt = x.T
    tail = tables[:, TAIL_START:, :].transpose(0, 2, 1)
    tail = tail.reshape(N_FIELDS, EMBED_DIM * TAIL)
    out_t = _emb_sum_t(tt, xt, tail)
    return out_t.T


# speedup vs baseline: 1.2670x; 1.0210x over previous
import functools

import jax
import jax.numpy as jnp
from jax import lax
from jax.experimental import pallas as pl
from jax.experimental.pallas import tpu as pltpu
from jax.experimental.pallas import tpu_sc as plsc

N_FIELDS = 26
VOCAB = 100000
EMBED_DIM = 32
BATCH = 16384

NUM_CORES = 2
NUM_SUBCORES = 16
IDX_CHUNK = 4096
N_IDX_CHUNKS = BATCH // IDX_CHUNK

SPLIT = 50048
TAIL_START = 99872
TAIL = 128
HALF_LENS = (SPLIT, VOCAB - SPLIT)
HALF_DMA_LENS = (SPLIT, 49920)

_mesh = plsc.VectorSubcoreMesh(
    core_axis_name="c", subcore_axis_name="s",
    num_cores=NUM_CORES, num_subcores=NUM_SUBCORES)


@functools.partial(
    pl.kernel,
    mesh=_mesh,
    out_type=jax.ShapeDtypeStruct((EMBED_DIM, BATCH), jnp.float32),
    scratch_types=[
        pltpu.VMEM((SPLIT,), jnp.float32),
        pltpu.VMEM((SPLIT,), jnp.float32),
        pltpu.VMEM((2, IDX_CHUNK), jnp.int32),
        pltpu.VMEM((BATCH,), jnp.float32),
        pltpu.VMEM((N_FIELDS, TAIL), jnp.float32),
        pltpu.VMEM_SHARED((2, BATCH), jnp.int32),
        pltpu.SemaphoreType.DMA,
        pltpu.SemaphoreType.DMA,
        pltpu.SemaphoreType.DMA,
        pltpu.SemaphoreType.DMA,
    ],
    compiler_params=pltpu.CompilerParams(use_tc_tiling_on_sc=True,
                                         needs_layout_passes=False),
)
def _emb_sum_t(tt_hbm, xt_hbm, tail_hbm, out_hbm, stripe0_v, stripe1_v,
               idx_v, acc_v, tail_v, idx_sh, sem_s0, sem_s1, sem_i, sem_b):
    sid = lax.axis_index("s")
    w = sid * NUM_CORES + lax.axis_index("c")
    d = w
    sems = (sem_s0, sem_s1)
    stripes = (stripe0_v, stripe1_v)

    def stripe_copies(f, h):
        off = h * SPLIT
        n = HALF_DMA_LENS[h]
        return [pltpu.make_async_copy(
            tt_hbm.at[f, :, pl.ds(off, n)].at[d],
            stripes[h].at[pl.ds(0, n)], sems[h])]

    def issue_stripe(f, h):
        for c in stripe_copies(f, h):
            c.start()

    def wait_stripe(f, h):
        for c in stripe_copies(f, h):
            c.wait()

    @pl.when(sid == 0)
    def _():
        pltpu.async_copy(xt_hbm.at[0], idx_sh.at[0], sem_b)

    def pull_copy(f, h, k):
        return pltpu.make_async_copy(
            idx_sh.at[f % 2, pl.ds(k * IDX_CHUNK, IDX_CHUNK)],
            idx_v.at[k % 2], sem_i)

    def half_pass(f, h):
        base = h * SPLIT
        bound = HALF_LENS[h]

        for k in range(N_IDX_CHUNKS):
            if k + 1 < N_IDX_CHUNKS:
                pull_copy(f, h, k + 1).start()
            pull_copy(f, h, k).wait()

            def vreg_body(i, k=k):
                iv = idx_v[k % 2, pl.ds(i * 16, 16)] - base
                m = plsc.bitcast(iv, jnp.uint32) < jnp.uint32(bound)
                g = plsc.load_gather(stripes[h], [iv], mask=m)
                g = jnp.where(m, g, jnp.float32(0.0))
                o = k * IDX_CHUNK + i * 16
                acc_v[pl.ds(o, 16)] = acc_v[pl.ds(o, 16)] + g

            plsc.parallel_loop(0, IDX_CHUNK // 16, unroll=8)(vreg_body)

    def zero_body(i):
        acc_v[pl.ds(i * 16, 16)] = jnp.zeros((16,), jnp.float32)

    plsc.parallel_loop(0, BATCH // 16, unroll=8)(zero_body)

    pltpu.sync_copy(tail_hbm.at[:, pl.ds(d * TAIL, TAIL)], tail_v)

    issue_stripe(0, 0)

    def field_body(f, carry):
        issue_stripe(f, 1)

        @pl.when(sid == 0)
        def _():
            pltpu.make_async_copy(xt_hbm.at[f], idx_sh.at[f % 2],
                                  sem_b).wait()

        plsc.subcore_barrier()

        @pl.when((sid == 0) & (f + 1 < N_FIELDS))
        def _():
            pltpu.async_copy(xt_hbm.at[f + 1], idx_sh.at[(f + 1) % 2], sem_b)

        wait_stripe(f, 0)
        pull_copy(f, 0, 0).start()
        half_pass(f, 0)

        wait_stripe(f, 1)

        @pl.when(f + 1 < N_FIELDS)
        def _():
            issue_stripe(f + 1, 0)

        for k in range(TAIL // 16):
            stripes[1][pl.ds(TAIL_START - SPLIT + k * 16, 16)] = (
                tail_v[f, pl.ds(k * 16, 16)])

        pull_copy(f, 1, 0).start()
        half_pass(f, 1)
        return carry

    lax.fori_loop(0, N_FIELDS, field_body, 0)

    pltpu.sync_copy(acc_v, out_hbm.at[d])


def kernel(x, tables):
    tt = tables.transpose(0, 2, 1)
    xt = x.T
    tail = tables[:, TAIL_START:, :].transpose(0, 2, 1)
    tail = tail.reshape(N_FIELDS, EMBED_DIM * TAIL)
    out_t = _emb_sum_t(tt, xt, tail)
    return out_t.T
